# Initial kernel scaffold; baseline (speedup 1.0000x reference)
import jax
import jax.numpy as jnp
from jax.experimental import pallas as pl


def kernel(x, adj_edge_index, adj_edge_weight, W_ih, W_hh, b_ih, b_hh, ln_gamma, ln_beta):
    raise NotImplementedError("write your pallas kernel here")


# speedup vs baseline: 4.4540x; 4.4540x over previous
import functools

import jax
import jax.numpy as jnp
from jax import lax
from jax.experimental import pallas as pl
from jax.experimental.pallas import tpu as pltpu
from jax.experimental.pallas import tpu_sc as plsc

N = 10000
E = 320000
K = 3
D = 128
H = 128

CH = 128
NCHUNK = E // CH
LANES = 16
ROW_CH = 128
NRC = N // ROW_CH
ROW_TAIL = N - NRC * ROW_CH


def _sc_body(x_hbm, idx_hbm, w_hbm, out_hbm, row_v, col_v, w_v, rows_v, zbuf, acc, sem):
    nc = 2
    ns = 16
    nw = nc * ns
    cid = lax.axis_index("c")
    sid = lax.axis_index("s")
    wid = sid * nc + cid

    def _zrow(r, carry):
        for p in range(ROW_CH // LANES):
            zbuf[r, pl.ds(p * LANES, LANES)] = jnp.zeros((LANES,), jnp.float32)
        return carry

    lax.fori_loop(0, ROW_CH, _zrow, 0)

    n_rounds = -(-NCHUNK // nw)
    n_zr = -(-(NRC + 1) // ns)

    for t in range(K):
        ka = K - 1 - t

        for i in range(n_zr):
            c = sid + ns * i

            @pl.when(c < NRC)
            def _():
                pltpu.sync_copy(zbuf, acc.at[pl.ds(c * ROW_CH, ROW_CH)])

            @pl.when(c == NRC)
            def _():
                pltpu.sync_copy(zbuf.at[pl.ds(0, ROW_TAIL)],
                                acc.at[pl.ds(NRC * ROW_CH, ROW_TAIL)])

        plsc.subcore_barrier()

        def _chunk(i, carry):
            c = wid + nw * i

            @pl.when(c < NCHUNK)
            def _():
                base = c * CH
                pltpu.sync_copy(idx_hbm.at[pl.ds((2 * ka) * E + base, CH)], row_v)
                pltpu.sync_copy(idx_hbm.at[pl.ds((2 * ka + 1) * E + base, CH)], col_v)
                pltpu.sync_copy(w_hbm.at[pl.ds(ka * E + base, CH)], w_v)
                pltpu.async_copy(x_hbm.at[col_v], rows_v, sem).wait()

                def _scale(g, cr):
                    wv = w_v[pl.ds(g * LANES, LANES)]
                    for l in range(LANES):
                        j = g * LANES + l
                        w = wv[l]
                        for p in range(D // LANES):
                            sl = pl.ds(p * LANES, LANES)
                            rows_v[j, sl] = rows_v[j, sl] * w
                    return cr

                lax.fori_loop(0, CH // LANES, _scale, 0)
                pltpu.sync_copy(rows_v, acc.at[row_v], add=True)

            return carry

        lax.fori_loop(0, n_rounds, _chunk, 0)
        plsc.subcore_barrier()

        off = (2 * t + cid) * N
        for i in range(n_zr):
            c = sid + ns * i

            @pl.when(c < NRC)
            def _():
                pltpu.sync_copy(acc.at[pl.ds(c * ROW_CH, ROW_CH)],
                                out_hbm.at[pl.ds(off + c * ROW_CH, ROW_CH)])

            @pl.when(c == NRC)
            def _():
                pltpu.sync_copy(acc.at[pl.ds(NRC * ROW_CH, ROW_TAIL)],
                                out_hbm.at[pl.ds(off + NRC * ROW_CH, ROW_TAIL)])

        plsc.subcore_barrier()


def _diffuse(x, idx32, w):
    mesh = plsc.VectorSubcoreMesh(core_axis_name="c", subcore_axis_name="s")
    fn = pl.kernel(
        _sc_body,
        mesh=mesh,
        out_type=jax.ShapeDtypeStruct((K * 2 * N, D), jnp.float32),
        scratch_types=[
            pltpu.VMEM((CH,), jnp.int32),
            pltpu.VMEM((CH,), jnp.int32),
            pltpu.VMEM((CH,), jnp.float32),
            pltpu.VMEM((CH, D), jnp.float32),
            pltpu.VMEM((ROW_CH, D), jnp.float32),
            pltpu.VMEM_SHARED((N, D), jnp.float32),
            pltpu.SemaphoreType.DMA,
        ],
    )
    return fn(x, idx32, w)


def _gru_body(hx_ref, wih_ref, whh_ref, bih_ref, bhh_ref, g_ref, b_ref, o_ref):
    R = o_ref.shape[0]
    h = jnp.zeros((R, H), jnp.float32)
    acc = jnp.zeros((R, H), jnp.float32)
    for t in range(K):
        xt = jnp.maximum(hx_ref[2 * t] + hx_ref[2 * t + 1], 0.0)
        gi = jnp.dot(xt, wih_ref[...], preferred_element_type=jnp.float32) + bih_ref[...]
        gh = jnp.dot(h, whh_ref[...], preferred_element_type=jnp.float32) + bhh_ref[...]
        r = jax.nn.sigmoid(gi[:, :H] + gh[:, :H])
        z = jax.nn.sigmoid(gi[:, H:2 * H] + gh[:, H:2 * H])
        n = jnp.tanh(gi[:, 2 * H:] + r * gh[:, 2 * H:])
        h = (1.0 - z) * n + z * h
        acc = acc + h
    mean = jnp.mean(acc, axis=1, keepdims=True)
    cen = acc - mean
    var = jnp.mean(cen * cen, axis=1, keepdims=True)
    o_ref[...] = cen * lax.rsqrt(var + 1e-5) * g_ref[...] + b_ref[...]


def _gru(hx, wih_t, whh_t, bih, bhh, g, b):
    R = 1000
    grid = (N // R,)
    return pl.pallas_call(
        _gru_body,
        grid=grid,
        in_specs=[
            pl.BlockSpec((2 * K, R, D), lambda i: (0, i, 0)),
            pl.BlockSpec((D, 3 * H), lambda i: (0, 0)),
            pl.BlockSpec((H, 3 * H), lambda i: (0, 0)),
            pl.BlockSpec((1, 3 * H), lambda i: (0, 0)),
            pl.BlockSpec((1, 3 * H), lambda i: (0, 0)),
            pl.BlockSpec((1, H), lambda i: (0, 0)),
            pl.BlockSpec((1, H), lambda i: (0, 0)),
        ],
        out_specs=pl.BlockSpec((R, H), lambda i: (i, 0)),
        out_shape=jax.ShapeDtypeStruct((N, H), jnp.float32),
    )(hx, wih_t, whh_t, bih, bhh, g, b)


def kernel(x, adj_edge_index, adj_edge_weight, W_ih, W_hh, b_ih, b_hh, ln_gamma, ln_beta):
    idx32 = adj_edge_index.astype(jnp.int32).reshape(-1)
    hx = _diffuse(x, idx32, adj_edge_weight.reshape(-1))
    hx = hx.reshape(2 * K, N, D)
    return _gru(hx, W_ih.T, W_hh.T, b_ih[None, :], b_hh[None, :],
                ln_gamma[None, :], ln_beta[None, :])


# speedup vs baseline: 5.2034x; 1.1683x over previous
import functools

import jax
import jax.numpy as jnp
from jax import lax
from jax.experimental import pallas as pl
from jax.experimental.pallas import tpu as pltpu
from jax.experimental.pallas import tpu_sc as plsc

N = 10000
E = 320000
K = 3
D = 128
H = 128

CH = 80
NW = 32
EPW = E // NW
NPW = EPW // CH
LANES = 16
ROW_CH = 128
NRC = N // ROW_CH
ROW_TAIL = N - NRC * ROW_CH


def _sc_body(x_hbm, idx_hbm, w_hbm, out_hbm,
             row_v0, col_v0, w_v0, rows_v0,
             row_v1, col_v1, w_v1, rows_v1,
             zbuf, acc, sem0, sem1):
    nc = 2
    ns = 16
    cid = lax.axis_index("c")
    sid = lax.axis_index("s")
    wid = sid * nc + cid

    def _zrow(r, carry):
        for p in range(ROW_CH // LANES):
            zbuf[r, pl.ds(p * LANES, LANES)] = jnp.zeros((LANES,), jnp.float32)
        return carry

    lax.fori_loop(0, ROW_CH, _zrow, 0)

    n_zr = -(-(NRC + 1) // ns)

    for t in range(K):
        ka = K - 1 - t

        for i in range(n_zr):
            c = sid + ns * i

            @pl.when(c < NRC)
            def _():
                pltpu.sync_copy(zbuf, acc.at[pl.ds(c * ROW_CH, ROW_CH)])

            @pl.when(c == NRC)
            def _():
                pltpu.sync_copy(zbuf.at[pl.ds(0, ROW_TAIL)],
                                acc.at[pl.ds(NRC * ROW_CH, ROW_TAIL)])

        plsc.subcore_barrier()

        def _fetch(j, row_r, col_r, w_r):
            base = wid * EPW + j * CH
            pltpu.sync_copy(idx_hbm.at[pl.ds((2 * ka) * E + base, CH)], row_r)
            pltpu.sync_copy(idx_hbm.at[pl.ds((2 * ka + 1) * E + base, CH)], col_r)
            pltpu.sync_copy(w_hbm.at[pl.ds(ka * E + base, CH)], w_r)

        def _gather_start(col_r, rows_r, sem):
            pltpu.async_copy(x_hbm.at[col_r], rows_r, sem)

        def _gather_wait(col_r, rows_r, sem):
            pltpu.make_async_copy(x_hbm.at[col_r], rows_r, sem).wait()

        def _scale(w_r, rows_r):
            def _grp(g, cr):
                wv = w_r[pl.ds(g * LANES, LANES)]
                for l in range(LANES):
                    j = g * LANES + l
                    w = wv[l]
                    for p in range(D // LANES):
                        sl = pl.ds(p * LANES, LANES)
                        rows_r[j, sl] = rows_r[j, sl] * w
                return cr

            lax.fori_loop(0, CH // LANES, _grp, 0)

        def _process(row_r, col_r, w_r, rows_r, sem):
            _gather_wait(col_r, rows_r, sem)
            _scale(w_r, rows_r)
            pltpu.sync_copy(rows_r, acc.at[row_r], add=True)

        _fetch(0, row_v0, col_v0, w_v0)
        _gather_start(col_v0, rows_v0, sem0)

        def _pair(i, carry):
            b = 2 * i + 1

            @pl.when(b < NPW)
            def _():
                _fetch(b, row_v1, col_v1, w_v1)
                _gather_start(col_v1, rows_v1, sem1)

            _process(row_v0, col_v0, w_v0, rows_v0, sem0)

            @pl.when(2 * i + 2 < NPW)
            def _():
                _fetch(2 * i + 2, row_v0, col_v0, w_v0)
                _gather_start(col_v0, rows_v0, sem0)

            @pl.when(b < NPW)
            def _():
                _process(row_v1, col_v1, w_v1, rows_v1, sem1)

            return carry

        lax.fori_loop(0, (NPW + 1) // 2, _pair, 0)
        plsc.subcore_barrier()

        off = (2 * t + cid) * N
        for i in range(n_zr):
            c = sid + ns * i

            @pl.when(c < NRC)
            def _():
                pltpu.sync_copy(acc.at[pl.ds(c * ROW_CH, ROW_CH)],
                                out_hbm.at[pl.ds(off + c * ROW_CH, ROW_CH)])

            @pl.when(c == NRC)
            def _():
                pltpu.sync_copy(acc.at[pl.ds(NRC * ROW_CH, ROW_TAIL)],
                                out_hbm.at[pl.ds(off + NRC * ROW_CH, ROW_TAIL)])

        plsc.subcore_barrier()


def _diffuse(x, idx32, w):
    mesh = plsc.VectorSubcoreMesh(core_axis_name="c", subcore_axis_name="s")
    fn = pl.kernel(
        _sc_body,
        mesh=mesh,
        out_type=jax.ShapeDtypeStruct((K * 2 * N, D), jnp.float32),
        scratch_types=[
            pltpu.VMEM((CH,), jnp.int32),
            pltpu.VMEM((CH,), jnp.int32),
            pltpu.VMEM((CH,), jnp.float32),
            pltpu.VMEM((CH, D), jnp.float32),
            pltpu.VMEM((CH,), jnp.int32),
            pltpu.VMEM((CH,), jnp.int32),
            pltpu.VMEM((CH,), jnp.float32),
            pltpu.VMEM((CH, D), jnp.float32),
            pltpu.VMEM((ROW_CH, D), jnp.float32),
            pltpu.VMEM_SHARED((N, D), jnp.float32),
            pltpu.SemaphoreType.DMA,
            pltpu.SemaphoreType.DMA,
        ],
    )
    return fn(x, idx32, w)


def _gru_body(hx_ref, wih_ref, whh_ref, bih_ref, bhh_ref, g_ref, b_ref, o_ref):
    R = o_ref.shape[0]
    h = jnp.zeros((R, H), jnp.float32)
    acc = jnp.zeros((R, H), jnp.float32)
    for t in range(K):
        xt = jnp.maximum(hx_ref[2 * t] + hx_ref[2 * t + 1], 0.0)
        gi = jnp.dot(xt, wih_ref[...], preferred_element_type=jnp.float32) + bih_ref[...]
        gh = jnp.dot(h, whh_ref[...], preferred_element_type=jnp.float32) + bhh_ref[...]
        r = jax.nn.sigmoid(gi[:, :H] + gh[:, :H])
        z = jax.nn.sigmoid(gi[:, H:2 * H] + gh[:, H:2 * H])
        n = jnp.tanh(gi[:, 2 * H:] + r * gh[:, 2 * H:])
        h = (1.0 - z) * n + z * h
        acc = acc + h
    mean = jnp.mean(acc, axis=1, keepdims=True)
    cen = acc - mean
    var = jnp.mean(cen * cen, axis=1, keepdims=True)
    o_ref[...] = cen * lax.rsqrt(var + 1e-5) * g_ref[...] + b_ref[...]


def _gru(hx, wih_t, whh_t, bih, bhh, g, b):
    R = 1000
    grid = (N // R,)
    return pl.pallas_call(
        _gru_body,
        grid=grid,
        in_specs=[
            pl.BlockSpec((2 * K, R, D), lambda i: (0, i, 0)),
            pl.BlockSpec((D, 3 * H), lambda i: (0, 0)),
            pl.BlockSpec((H, 3 * H), lambda i: (0, 0)),
            pl.BlockSpec((1, 3 * H), lambda i: (0, 0)),
            pl.BlockSpec((1, 3 * H), lambda i: (0, 0)),
            pl.BlockSpec((1, H), lambda i: (0, 0)),
            pl.BlockSpec((1, H), lambda i: (0, 0)),
        ],
        out_specs=pl.BlockSpec((R, H), lambda i: (i, 0)),
        out_shape=jax.ShapeDtypeStruct((N, H), jnp.float32),
    )(hx, wih_t, whh_t, bih, bhh, g, b)


def kernel(x, adj_edge_index, adj_edge_weight, W_ih, W_hh, b_ih, b_hh, ln_gamma, ln_beta):
    idx32 = adj_edge_index.astype(jnp.int32).reshape(-1)
    hx = _diffuse(x, idx32, adj_edge_weight.reshape(-1))
    hx = hx.reshape(2 * K, N, D)
    return _gru(hx, W_ih.T, W_hh.T, b_ih[None, :], b_hh[None, :],
                ln_gamma[None, :], ln_beta[None, :])
